# Initial kernel scaffold; baseline (speedup 1.0000x reference)
import jax
import jax.numpy as jnp
from jax.experimental import pallas as pl


def kernel(z, loc, log_scale, W1, b1, W2, b2, eps_rand):
    raise NotImplementedError("write your pallas kernel here")


# speedup vs baseline: 1.1424x; 1.1424x over previous
import functools

import numpy as np
import jax
import jax.numpy as jnp
from jax.experimental import pallas as pl
from jax.experimental.pallas import tpu as pltpu

_T = 100


def _main_kernel(z_ref, er_ref, loc_ref, ls_ref, w1_ref, b1_ref, w2t_ref,
                 b2_ref, acc_ref, lpg_ref, zsum_ref):
    i = pl.program_id(0)
    d = z_ref.shape[1]
    loc = loc_ref[...]
    ls = ls_ref[...]
    inv_scale = jnp.exp(-ls)
    c0 = -0.5 * d * np.log(2.0 * np.pi) - jnp.sum(ls)
    w1 = w1_ref[...]
    b1 = b1_ref[...]
    w2t = w2t_ref[...]
    b2 = b2_ref[0, 0]

    eps_z = (z_ref[...] - loc) * inv_scale
    lpg_ref[...] = c0 - 0.5 * jnp.sum(eps_z * eps_z, axis=1, keepdims=True)
    h = jnp.tanh(
        jnp.dot(eps_z.astype(jnp.bfloat16), w1,
                preferred_element_type=jnp.float32) + b1)
    logit = jnp.sum(h * w2t, axis=1, keepdims=True) + b2
    acc_ref[...] = jax.nn.sigmoid(logit)

    eps_r = (er_ref[...] - loc) * inv_scale
    hr = jnp.tanh(
        jnp.dot(eps_r.astype(jnp.bfloat16), w1,
                preferred_element_type=jnp.float32) + b1)
    logit_r = jnp.sum(hr * w2t, axis=1, keepdims=True) + b2
    zpart = jnp.sum(jax.nn.sigmoid(logit_r)).reshape(1, 1)

    @pl.when(i == 0)
    def _init():
        zsum_ref[...] = zpart

    @pl.when(i != 0)
    def _acc():
        zsum_ref[...] += zpart


def _combine_kernel(acc_ref, lpg_ref, zsum_ref, out_ref, *, n_total):
    Z = zsum_ref[0, 0] / n_total
    alpha = (1.0 - Z) ** (_T - 1)
    out_ref[...] = jnp.log((1.0 - alpha) * acc_ref[...] / Z + alpha) \
        + lpg_ref[...]


def kernel(z, loc, log_scale, W1, b1, W2, b2, eps_rand):
    B, D = z.shape
    H = W1.shape[1]
    bm = min(512, B)
    nb = B // bm

    w1_bf16 = W1.astype(jnp.bfloat16)
    b1_2d = b1.reshape(1, H)
    w2t = W2.reshape(1, H)
    b2_2d = b2.reshape(1, 1)

    acc, lpg, zsum = pl.pallas_call(
        _main_kernel,
        grid=(nb,),
        in_specs=[
            pl.BlockSpec((bm, D), lambda i: (i, 0)),
            pl.BlockSpec((bm, D), lambda i: (i, 0)),
            pl.BlockSpec((1, D), lambda i: (0, 0)),
            pl.BlockSpec((1, D), lambda i: (0, 0)),
            pl.BlockSpec((D, H), lambda i: (0, 0)),
            pl.BlockSpec((1, H), lambda i: (0, 0)),
            pl.BlockSpec((1, H), lambda i: (0, 0)),
            pl.BlockSpec((1, 1), lambda i: (0, 0)),
        ],
        out_specs=[
            pl.BlockSpec((bm, 1), lambda i: (i, 0)),
            pl.BlockSpec((bm, 1), lambda i: (i, 0)),
            pl.BlockSpec((1, 1), lambda i: (0, 0)),
        ],
        out_shape=[
            jax.ShapeDtypeStruct((B, 1), jnp.float32),
            jax.ShapeDtypeStruct((B, 1), jnp.float32),
            jax.ShapeDtypeStruct((1, 1), jnp.float32),
        ],
        compiler_params=pltpu.CompilerParams(
            dimension_semantics=("arbitrary",)),
    )(z, eps_rand, loc, log_scale, w1_bf16, b1_2d, w2t, b2_2d)

    log_p = pl.pallas_call(
        functools.partial(_combine_kernel, n_total=float(B)),
        out_shape=jax.ShapeDtypeStruct((B, 1), jnp.float32),
    )(acc, lpg, zsum)
    return log_p
